# baseline (speedup 1.0000x reference)
import functools
import math

import jax
import jax.numpy as jnp
from jax import lax
from jax.experimental import pallas as pl
from jax.experimental.pallas import tpu as pltpu
from jax.experimental.pallas import tpu_sc as plsc

D_MODEL = 64
CHUNK = 128
SCALE = math.sqrt(D_MODEL)


@functools.partial(jax.jit, static_argnames=("n_rows",))
def _embed(idx, table, n_rows):
    info = plsc.get_sparse_core_info()
    nw = info.num_cores * info.num_subcores
    b_per_w = n_rows // nw
    n_chunks = b_per_w // CHUNK
    mesh = plsc.VectorSubcoreMesh(core_axis_name="c", subcore_axis_name="s")

    @functools.partial(
        pl.kernel,
        mesh=mesh,
        compiler_params=pltpu.CompilerParams(use_tc_tiling_on_sc=False),
        out_type=jax.ShapeDtypeStruct((n_rows, D_MODEL), jnp.float32),
        scratch_types=[
            pltpu.VMEM((n_chunks, CHUNK), jnp.int32),
            pltpu.VMEM((CHUNK, D_MODEL), jnp.float32),
            pltpu.SemaphoreType.DMA,
        ],
    )
    def k(idx_hbm, table_hbm, out_hbm, idx_v, rows_v, sem):
        cid = lax.axis_index("c")
        sid = lax.axis_index("s")
        wid = sid * info.num_cores + cid
        base = wid * b_per_w
        pltpu.sync_copy(idx_hbm.at[wid], idx_v)

        def chunk_body(c, carry):
            pltpu.async_copy(table_hbm.at[idx_v.at[c]], rows_v, sem).wait()

            def mul_row(i, carry2):
                for v in range(D_MODEL // 16):
                    sl = pl.ds(v * 16, 16)
                    rows_v[i, sl] = rows_v[i, sl] * SCALE
                return carry2

            lax.fori_loop(0, CHUNK, mul_row, 0)
            pltpu.sync_copy(rows_v, out_hbm.at[pl.ds(base + c * CHUNK, CHUNK)])
            return carry

        lax.fori_loop(0, n_chunks, chunk_body, 0)

    return k(idx.reshape(nw, n_chunks, CHUNK), table)


def kernel(x, table):
    n_rows = x.size
    idx = x.reshape(-1).astype(jnp.int32)
    out = _embed(idx, table, n_rows)
    return out.reshape(*x.shape, D_MODEL)


# speedup vs baseline: 1.2082x; 1.2082x over previous
import functools
import math

import jax
import jax.numpy as jnp
from jax import lax
from jax.experimental import pallas as pl
from jax.experimental.pallas import tpu as pltpu
from jax.experimental.pallas import tpu_sc as plsc

D_MODEL = 64
CHUNK = 128
NBUF = 4
SCALE = math.sqrt(D_MODEL)


def _scale_chunk(buf):
    def body(i, carry):
        r0 = i * 8
        for r in range(8):
            for v in range(D_MODEL // 16):
                sl = pl.ds(v * 16, 16)
                buf[r0 + r, sl] = buf[r0 + r, sl] * SCALE
        return carry

    lax.fori_loop(0, CHUNK // 8, body, 0, unroll=False)


@functools.partial(jax.jit, static_argnames=("n_rows",))
def _embed(idx, table, n_rows):
    info = plsc.get_sparse_core_info()
    nw = info.num_cores * info.num_subcores
    b_per_w = n_rows // nw
    n_chunks = b_per_w // CHUNK
    n_groups = n_chunks // NBUF
    assert n_groups % 2 == 0
    mesh = plsc.VectorSubcoreMesh(core_axis_name="c", subcore_axis_name="s")

    @functools.partial(
        pl.kernel,
        mesh=mesh,
        compiler_params=pltpu.CompilerParams(use_tc_tiling_on_sc=False),
        out_type=jax.ShapeDtypeStruct((n_rows, D_MODEL), jnp.float32),
        scratch_types=[
            pltpu.VMEM((n_chunks, CHUNK), jnp.int32),
        ]
        + [pltpu.VMEM((CHUNK, D_MODEL), jnp.float32) for _ in range(2 * NBUF)]
        + [pltpu.SemaphoreType.DMA for _ in range(4)],
    )
    def k(idx_hbm, table_hbm, out_hbm, idx_v, *rest):
        bufs_a = rest[0:NBUF]
        bufs_b = rest[NBUF : 2 * NBUF]
        gsem_a, ssem_a, gsem_b, ssem_b = rest[2 * NBUF : 2 * NBUF + 4]

        cid = lax.axis_index("c")
        sid = lax.axis_index("s")
        wid = sid * info.num_cores + cid
        base = wid * b_per_w
        pltpu.sync_copy(idx_hbm.at[wid], idx_v)

        def gather(c, buf, sem):
            pltpu.async_copy(table_hbm.at[idx_v.at[c]], buf, sem)

        def gather_wait(c, buf, sem):
            pltpu.make_async_copy(table_hbm.at[idx_v.at[c]], buf, sem).wait()

        def scatter(c, buf, sem):
            dst = out_hbm.at[pl.ds(base + c * CHUNK, CHUNK)]
            pltpu.async_copy(buf, dst, sem)

        def scatter_wait(c, buf, sem):
            dst = out_hbm.at[pl.ds(base + c * CHUNK, CHUNK)]
            pltpu.make_async_copy(buf, dst, sem).wait()

        for b in range(NBUF):
            gather(b, bufs_a[b], gsem_a)

        def pair_body(p, carry):
            ga = 2 * p
            gb = 2 * p + 1

            for b in range(NBUF):
                gather(gb * NBUF + b, bufs_b[b], gsem_b)

            for b in range(NBUF):
                c = ga * NBUF + b
                gather_wait(c, bufs_a[b], gsem_a)
                _scale_chunk(bufs_a[b])
                scatter(c, bufs_a[b], ssem_a)

            for b in range(NBUF):
                c = ga * NBUF + b
                scatter_wait(c, bufs_a[b], ssem_a)

            @pl.when(ga + 2 < n_groups)
            def _():
                for b in range(NBUF):
                    gather((ga + 2) * NBUF + b, bufs_a[b], gsem_a)

            for b in range(NBUF):
                c = gb * NBUF + b
                gather_wait(c, bufs_b[b], gsem_b)
                _scale_chunk(bufs_b[b])
                scatter(c, bufs_b[b], ssem_b)

            for b in range(NBUF):
                c = gb * NBUF + b
                scatter_wait(c, bufs_b[b], ssem_b)

            return carry

        lax.fori_loop(0, n_groups // 2, pair_body, 0)

    return k(idx.reshape(nw, n_chunks, CHUNK), table)


def kernel(x, table):
    n_rows = x.size
    idx = x.reshape(-1).astype(jnp.int32)
    out = _embed(idx, table, n_rows)
    return out.reshape(*x.shape, D_MODEL)
